# baseline (speedup 1.0000x reference)
import functools

import jax
import jax.numpy as jnp
from jax.experimental import pallas as pl
from jax.experimental.pallas import tpu as pltpu

N, DIM, HID, MID, NB = 32768, 128, 64, 96, 16
BLK = 2048


def _mlp_body(x_ref, wr_ref, br_ref, w1_ref, b1_ref, g_ref, bb_ref, w2_ref,
              b2_ref, y_ref, mag_ref):
    x = x_ref[...]
    mag_ref[...] = jnp.dot(x, wr_ref[...],
                           preferred_element_type=jnp.float32) + br_ref[0]
    h = jnp.dot(x, w1_ref[...], preferred_element_type=jnp.float32) + b1_ref[...]
    mu = jnp.mean(h, axis=-1, keepdims=True)
    var = jnp.mean(jnp.square(h - mu), axis=-1, keepdims=True)
    hn = (h - mu) * jax.lax.rsqrt(var + 1e-5) * g_ref[...] + bb_ref[...]
    act = hn * jnp.tanh(jax.nn.softplus(hn))
    y_ref[...] = jnp.dot(act, w2_ref[...],
                         preferred_element_type=jnp.float32) + b2_ref[...]


@jax.jit
def _fused_mlp(x, W_rank, b_rank, W1, b1, ln_g, ln_b, W2, b2):
    grid = (N // BLK,)
    full = lambda shp: pl.BlockSpec(shp, lambda i: tuple(0 for _ in shp))
    y, mag = pl.pallas_call(
        _mlp_body,
        grid=grid,
        in_specs=[
            pl.BlockSpec((BLK, DIM), lambda i: (i, 0)),
            full((DIM, 1)),
            full((1,)),
            full((DIM, MID)),
            full((MID,)),
            full((MID,)),
            full((MID,)),
            full((MID, HID)),
            full((HID,)),
        ],
        out_specs=[
            pl.BlockSpec((BLK, HID), lambda i: (i, 0)),
            pl.BlockSpec((BLK, 1), lambda i: (i, 0)),
        ],
        out_shape=[
            jax.ShapeDtypeStruct((N, HID), jnp.float32),
            jax.ShapeDtypeStruct((N, 1), jnp.float32),
        ],
        compiler_params=pltpu.CompilerParams(
            dimension_semantics=("arbitrary",)),
    )(x, W_rank, b_rank, W1, b1, ln_g, ln_b, W2, b2)
    return y, mag


def kernel(x, batch, n_batches, W_rank, b_rank, W1, b1, ln_g, ln_b, W2, b2,
           key_table, W_card, b_card):
    batch = batch.astype(jnp.int32)
    y0, mag2 = _fused_mlp(x, W_rank, b_rank, W1, b1, ln_g, ln_b, W2, b2)
    mag = mag2.reshape(-1)
    perm = jnp.lexsort((mag, batch))
    n = jax.ops.segment_sum(jnp.ones((N,), jnp.float32), batch,
                            num_segments=NB)
    ptr = jnp.concatenate([jnp.zeros((1,), jnp.float32),
                           jnp.cumsum(n)]).astype(jnp.int32)
    k = jnp.arange(N, dtype=jnp.int32) - ptr[batch]
    rank = jnp.zeros((N,), jnp.int32).at[perm].set(k)
    keys = jnp.take(key_table, rank, axis=0)
    z_el = jax.ops.segment_sum(y0 * keys, batch, num_segments=NB)
    n_enc = n[:, None] @ W_card + b_card
    return z_el + n_enc


# speedup vs baseline: 2.0484x; 2.0484x over previous
import functools

import jax
import jax.numpy as jnp
from jax import lax
from jax.experimental import pallas as pl
from jax.experimental.pallas import tpu as pltpu
from jax.experimental.pallas import tpu_sc as plsc

N, DIM, HID, MID, NB = 32768, 128, 64, 96, 16
BLK = 2048

W = 16
CH = N // W
NV = CH // 16
RADIX = 32
NPASS = 8

_sort_mesh = plsc.VectorSubcoreMesh(core_axis_name="c", subcore_axis_name="s",
                                    num_cores=1, num_subcores=W)
_gath_mesh = plsc.VectorSubcoreMesh(core_axis_name="c", subcore_axis_name="s")
_GW = 32
_GCH = N // _GW


def _iota():
    return lax.iota(jnp.int32, 16)


def _mlp_body(x_ref, wr_ref, br_ref, w1_ref, b1_ref, g_ref, bb_ref, w2_ref,
              b2_ref, kt_ref, y_ref, mag_ref, ktp_ref):
    x = x_ref[...]
    mag_ref[...] = jnp.dot(x, wr_ref[...],
                           preferred_element_type=jnp.float32) + br_ref[0]
    h = jnp.dot(x, w1_ref[...], preferred_element_type=jnp.float32) + b1_ref[...]
    mu = jnp.mean(h, axis=-1, keepdims=True)
    var = jnp.mean(jnp.square(h - mu), axis=-1, keepdims=True)
    hn = (h - mu) * lax.rsqrt(var + 1e-5) * g_ref[...] + bb_ref[...]
    act = hn * jnp.tanh(jax.nn.softplus(hn))
    y_ref[...] = jnp.dot(act, w2_ref[...],
                         preferred_element_type=jnp.float32) + b2_ref[...]
    kt = kt_ref[...]
    ktp_ref[...] = jnp.concatenate([kt, kt], axis=1)


def _fused_mlp(x, W_rank, b_rank, W1, b1, ln_g, ln_b, W2, b2, key_table):
    full = lambda shp: pl.BlockSpec(shp, lambda i: tuple(0 for _ in shp))
    return pl.pallas_call(
        _mlp_body,
        grid=(N // BLK,),
        in_specs=[
            pl.BlockSpec((BLK, DIM), lambda i: (i, 0)),
            full((DIM, 1)), full((1,)), full((DIM, MID)), full((MID,)),
            full((MID,)), full((MID,)), full((MID, HID)), full((HID,)),
            pl.BlockSpec((BLK, HID), lambda i: (i, 0)),
        ],
        out_specs=[
            pl.BlockSpec((BLK, HID), lambda i: (i, 0)),
            pl.BlockSpec((BLK, 1), lambda i: (i, 0)),
            pl.BlockSpec((BLK, 2 * HID), lambda i: (i, 0)),
        ],
        out_shape=[
            jax.ShapeDtypeStruct((N, HID), jnp.float32),
            jax.ShapeDtypeStruct((N, 1), jnp.float32),
            jax.ShapeDtypeStruct((N, 2 * HID), jnp.float32),
        ],
        compiler_params=pltpu.CompilerParams(
            dimension_semantics=("arbitrary",)),
    )(x, W_rank, b_rank, W1, b1, ln_g, ln_b, W2, b2, key_table)


def _rank_body(mag_hbm, batch_hbm, rank_hbm,
               keyf_vm, key_vm, pay_vm, pos_vm, rank_vm, bat_vm,
               hist_vm, cur_vm, ptr_vm, grid_vm, excl_vm, shared_hist,
               shk0, shk1, shp0, shp1, shrank, semA, semB):
    sid = lax.axis_index("s")
    base = sid * CH
    key_bufs = [shk0, shk1]
    pay_bufs = [shp0, shp1]

    for p in range(NPASS):
        sh = 5 * p

        if p == 0:
            pltpu.sync_copy(mag_hbm.at[pl.ds(base, CH)], keyf_vm)
            pltpu.sync_copy(batch_hbm.at[pl.ds(base, CH)], bat_vm)

            def init_body(v, c):
                f = keyf_vm[pl.ds(v * 16, 16)]
                u = plsc.bitcast(f, jnp.uint32)
                neg = u >> 31
                flip = (jnp.uint32(0) - neg) | jnp.uint32(0x80000000)
                key_vm[pl.ds(v * 16, 16)] = plsc.bitcast(
                    (u ^ flip) ^ jnp.uint32(0x80000000), jnp.int32)
                idx = base + v * 16 + _iota()
                pay_vm[pl.ds(v * 16, 16)] = idx | (
                    bat_vm[pl.ds(v * 16, 16)] << 16)
                return c
            lax.fori_loop(0, NV, init_body, 0)
        else:
            pltpu.sync_copy(key_bufs[(p + 1) % 2].at[pl.ds(base, CH)], key_vm)
            pltpu.sync_copy(pay_bufs[(p + 1) % 2].at[pl.ds(base, CH)], pay_vm)

        z16 = _iota() * 0
        hist_vm[pl.ds(0, 16)] = z16
        hist_vm[pl.ds(16, 16)] = z16

        def hist_body(v, c):
            if p < 7:
                kv = plsc.bitcast(key_vm[pl.ds(v * 16, 16)], jnp.uint32)
                d = (((kv ^ jnp.uint32(0x80000000)) >> jnp.uint32(sh))
                     & jnp.uint32(31)).astype(jnp.int32)
            else:
                d = pay_vm[pl.ds(v * 16, 16)] >> 16
            cnt, last = plsc.scan_count(d)
            plsc.addupdate_scatter(hist_vm, [d], cnt, mask=last)
            return c
        lax.fori_loop(0, NV, hist_body, 0)

        pltpu.sync_copy(hist_vm, shared_hist.at[pl.ds(sid * RADIX, RADIX)])
        plsc.subcore_barrier()

        pltpu.sync_copy(shared_hist, grid_vm)

        def tr_body(j, c):
            col = plsc.load_gather(grid_vm, [_iota() * RADIX + j])
            excl_vm[pl.ds(j * 16, 16)] = col
            return c
        lax.fori_loop(0, RADIX, tr_body, 0)

        def scan_body(j, carry):
            v = excl_vm[pl.ds(j * 16, 16)]
            ex = plsc.cumsum(v) - v + carry
            excl_vm[pl.ds(j * 16, 16)] = ex
            return carry + jnp.sum(v)
        lax.fori_loop(0, RADIX, scan_body, jnp.int32(0))

        cur_vm[pl.ds(0, 16)] = plsc.load_gather(excl_vm, [_iota() * 16 + sid])
        cur_vm[pl.ds(16, 16)] = plsc.load_gather(
            excl_vm, [(_iota() + 16) * 16 + sid])
        if p == NPASS - 1:
            ptr_vm[pl.ds(0, 16)] = plsc.load_gather(excl_vm, [_iota() * 16])
            ptr_vm[pl.ds(16, 16)] = plsc.load_gather(
                excl_vm, [(_iota() + 16) * 16])

        def perm_body(v, c):
            if p < 7:
                kv = plsc.bitcast(key_vm[pl.ds(v * 16, 16)], jnp.uint32)
                d = (((kv ^ jnp.uint32(0x80000000)) >> jnp.uint32(sh))
                     & jnp.uint32(31)).astype(jnp.int32)
            else:
                d = pay_vm[pl.ds(v * 16, 16)] >> 16
            cnt, last = plsc.scan_count(d)
            basev = plsc.load_gather(cur_vm, [d])
            pos = basev + cnt - 1
            row = v >> 3
            col = (v & 7) * 16
            if p == NPASS - 1:
                rank_vm[pl.ds(v * 16, 16)] = pos - plsc.load_gather(ptr_vm, [d])
                pos_vm[row, pl.ds(col, 16)] = pay_vm[pl.ds(v * 16, 16)] & 0xFFFF
            else:
                pos_vm[row, pl.ds(col, 16)] = pos
            plsc.addupdate_scatter(cur_vm, [d], cnt, mask=last)
            return c
        lax.fori_loop(0, NV, perm_body, 0)

        copies = []
        if p < NPASS - 1:
            dst = p % 2
            for j in range(W):
                copies.append(pltpu.async_copy(
                    key_vm.at[pl.ds(j * 128, 128)],
                    key_bufs[dst].at[pos_vm.at[j]], semA))
                copies.append(pltpu.async_copy(
                    pay_vm.at[pl.ds(j * 128, 128)],
                    pay_bufs[dst].at[pos_vm.at[j]], semB))
        else:
            for j in range(W):
                copies.append(pltpu.async_copy(
                    rank_vm.at[pl.ds(j * 128, 128)],
                    shrank.at[pos_vm.at[j]], semA))
        for c in copies:
            c.wait()
        plsc.subcore_barrier()

    pltpu.sync_copy(shrank.at[pl.ds(base, CH)],
                    rank_hbm.at[pl.ds(base, CH)])


def _rank_op(mag, batch):
    outs = pl.kernel(
        _rank_body,
        out_type=jax.ShapeDtypeStruct((N,), jnp.int32),
        mesh=_sort_mesh,
        scratch_types=[
            pltpu.VMEM((CH,), jnp.float32),
            pltpu.VMEM((CH,), jnp.int32),
            pltpu.VMEM((CH,), jnp.int32),
            pltpu.VMEM((W, 128), jnp.int32),
            pltpu.VMEM((CH,), jnp.int32),
            pltpu.VMEM((CH,), jnp.int32),
            pltpu.VMEM((RADIX,), jnp.int32),
            pltpu.VMEM((RADIX,), jnp.int32),
            pltpu.VMEM((RADIX,), jnp.int32),
            pltpu.VMEM((W * RADIX,), jnp.int32),
            pltpu.VMEM((W * RADIX,), jnp.int32),
            pltpu.VMEM_SHARED((W * RADIX,), jnp.int32),
            pltpu.VMEM_SHARED((N,), jnp.int32),
            pltpu.VMEM_SHARED((N,), jnp.int32),
            pltpu.VMEM_SHARED((N,), jnp.int32),
            pltpu.VMEM_SHARED((N,), jnp.int32),
            pltpu.VMEM_SHARED((N,), jnp.int32),
            pltpu.SemaphoreType.DMA,
            pltpu.SemaphoreType.DMA,
        ],
        compiler_params=pltpu.CompilerParams(needs_layout_passes=False),
    )(mag, batch)
    return outs


_GR = _GCH // 128


def _gather_body(table_hbm, idx_hbm, out_hbm, idx_vm, rows_vm, sem):
    wid = lax.axis_index("s") * 2 + lax.axis_index("c")
    row0 = wid * _GR
    pltpu.sync_copy(idx_hbm.at[pl.ds(row0, _GR)], idx_vm)
    for half in range(2):
        copies = []
        for j in range(_GR // 2):
            copies.append(pltpu.async_copy(
                table_hbm.at[idx_vm.at[half * (_GR // 2) + j]],
                rows_vm.at[pl.ds(j * 128, 128)], sem))
        for c in copies:
            c.wait()
        pltpu.sync_copy(
            rows_vm,
            out_hbm.at[pl.ds(row0 * 128 + half * (_GCH // 2), _GCH // 2)])


def _gather_op(table, idx2):
    return pl.kernel(
        _gather_body,
        out_type=jax.ShapeDtypeStruct((N, 2 * HID), jnp.float32),
        mesh=_gath_mesh,
        scratch_types=[
            pltpu.VMEM((_GR, 128), jnp.int32),
            pltpu.VMEM((_GCH // 2, 2 * HID), jnp.float32),
            pltpu.SemaphoreType.DMA,
        ],
        compiler_params=pltpu.CompilerParams(needs_layout_passes=False),
    )(table, idx2)


def _reduce_body(y_ref, k_ref, b_ref, wc_ref, bc_ref, z_ref, n_ref, nsteps):
    i = pl.program_id(0)

    @pl.when(i == 0)
    def _():
        z_ref[...] = jnp.zeros_like(z_ref)
        n_ref[...] = jnp.zeros_like(n_ref)

    prod = y_ref[...] * k_ref[:, :HID]
    seg = b_ref[0, 0, :].reshape(1, BLK)
    ids = lax.broadcasted_iota(jnp.int32, (NB, BLK), 0)
    onehot = (seg == ids).astype(jnp.float32)
    z_ref[...] += jnp.dot(onehot, prod, preferred_element_type=jnp.float32,
                          precision=lax.Precision.HIGHEST)
    n_ref[...] += jnp.sum(onehot, axis=1, keepdims=True)

    @pl.when(i == nsteps - 1)
    def _():
        z_ref[...] += n_ref[...] * wc_ref[...] + bc_ref[...]


def _reduce_op(y0, keys_g, batch3, W_card, b_card):
    nsteps = N // BLK
    full = lambda shp: pl.BlockSpec(shp, lambda i: tuple(0 for _ in shp))
    z, _ = pl.pallas_call(
        functools.partial(_reduce_body, nsteps=nsteps),
        grid=(nsteps,),
        in_specs=[
            pl.BlockSpec((BLK, HID), lambda i: (i, 0)),
            pl.BlockSpec((BLK, 2 * HID), lambda i: (i, 0)),
            pl.BlockSpec((1, 1, BLK), lambda i: (i, 0, 0)),
            full((1, HID)), full((HID,)),
        ],
        out_specs=[
            pl.BlockSpec((NB, HID), lambda i: (0, 0)),
            pl.BlockSpec((NB, 1), lambda i: (0, 0)),
        ],
        out_shape=[
            jax.ShapeDtypeStruct((NB, HID), jnp.float32),
            jax.ShapeDtypeStruct((NB, 1), jnp.float32),
        ],
        compiler_params=pltpu.CompilerParams(
            dimension_semantics=("arbitrary",)),
    )(y0, keys_g, batch3, W_card, b_card)
    return z


def kernel(x, batch, n_batches, W_rank, b_rank, W1, b1, ln_g, ln_b, W2, b2,
           key_table, W_card, b_card):
    batch = batch.astype(jnp.int32)
    y0, mag2, ktp = _fused_mlp(x, W_rank, b_rank, W1, b1, ln_g, ln_b, W2,
                               b2, key_table)
    rank = _rank_op(mag2.reshape(-1), batch)
    keys_g = _gather_op(ktp, rank.reshape(N // 128, 128))
    return _reduce_op(y0, keys_g, batch.reshape(N // BLK, 1, BLK), W_card,
                      b_card)
